# baseline (device time: 147651 ns/iter reference)
import jax
import jax.numpy as jnp
from jax import lax
from jax.experimental import pallas as pl
from jax.experimental.pallas import tpu as pltpu

N_DEV = 4
K_CHUNK = 2048
X_TILE = 512
HALF = 1024
DEST_OFFSETS = (2, 1, 3, 0)
N_JOBS = 2 * N_DEV


def _gelu(y):
    c = 0.7978845608028654
    return 0.5 * y * (1.0 + jnp.tanh(c * (y + 0.044715 * y * y * y)))


def kernel(x, w_mat):
    m_per, k_dim = x.shape
    _, n_dim = w_mat.shape
    n_per = n_dim // N_DEV
    n_c = k_dim // K_CHUNK
    n_xt = k_dim // X_TILE
    n_tiles = N_JOBS * n_c

    def body(x_ref, w_ref, out_ref, xstage_ref, xbf_ref, wtile_ref,
             acc_ref, stage_ref, send_ref, recv_ref,
             xdma_sems, wdma_sems, send_sems, recv_sems, out_sems):
        my = lax.axis_index("i")

        barrier_sem = pltpu.get_barrier_semaphore()
        for off in (1, 2, 3):
            pl.semaphore_signal(
                barrier_sem, inc=1,
                device_id=((my + off) % N_DEV,),
                device_id_type=pl.DeviceIdType.MESH,
            )

        def x_dma(i, slot):
            return pltpu.make_async_copy(
                x_ref.at[:, pl.ds(i * X_TILE, X_TILE)],
                xstage_ref.at[slot],
                xdma_sems.at[slot],
            )

        def w_dma(t):
            jj, c = t // n_c, t % n_c
            d = (my + DEST_OFFSETS[jj // 2]) % N_DEV
            col = d * n_per + (jj % 2) * HALF
            return pltpu.make_async_copy(
                w_ref.at[pl.ds(c * K_CHUNK, K_CHUNK), pl.ds(col, HALF)],
                wtile_ref.at[t % 2],
                wdma_sems.at[t % 2],
            )

        x_dma(0, 0).start()
        w_dma(0).start()
        w_dma(1).start()
        for i in range(n_xt):
            if i + 1 < n_xt:
                x_dma(i + 1, (i + 1) % 2).start()
            x_dma(i, i % 2).wait()
            xbf_ref[:, i * X_TILE:(i + 1) * X_TILE] = (
                xstage_ref[i % 2].astype(jnp.bfloat16))

        def chunk_dot(c, t):
            return jnp.dot(
                xbf_ref[:, c * K_CHUNK:(c + 1) * K_CHUNK],
                wtile_ref[t % 2].astype(jnp.bfloat16),
                preferred_element_type=jnp.float32)

        rdmas = []
        own_cps = []

        def compute_job(jj):
            h = jj % 2
            d = (my + DEST_OFFSETS[jj // 2]) % N_DEV
            base = jj * n_c
            w_dma(base).wait()
            acc_ref[...] = chunk_dot(0, base)
            if base + 2 < n_tiles:
                w_dma(base + 2).start()
            w_dma(base + 1).wait()
            y = _gelu(acc_ref[...] + chunk_dot(1, base + 1))
            if jj < 6:
                sslot = jj % 4
                if jj >= 4:
                    rdmas[jj - 4].wait_send()
                send_ref[sslot] = y.astype(jnp.bfloat16)
                if base + 3 < n_tiles:
                    w_dma(base + 3).start()
                if jj == 0:
                    pl.semaphore_wait(barrier_sem, N_DEV - 1)
                rslot = (my - d - 1) % N_DEV
                rdma = pltpu.make_async_remote_copy(
                    src_ref=send_ref.at[sslot],
                    dst_ref=recv_ref.at[rslot, :, pl.ds(h * HALF, HALF)],
                    send_sem=send_sems.at[jj],
                    recv_sem=recv_sems.at[2 * rslot + h],
                    device_id=(d,),
                    device_id_type=pl.DeviceIdType.MESH,
                )
                rdma.start()
                rdmas.append(rdma)
            else:
                acc_ref[...] = y
                if base + 3 < n_tiles:
                    w_dma(base + 3).start()
                cp = pltpu.make_async_copy(
                    acc_ref,
                    out_ref.at[pl.ds(my * m_per, m_per), pl.ds(h * HALF, HALF)],
                    out_sems.at[h])
                cp.start()
                own_cps.append(cp)

        state = {"cp": None}

        def drain(j, h):
            src = (my + 1 + j) % N_DEV
            pltpu.make_async_remote_copy(
                src_ref=send_ref.at[0],
                dst_ref=recv_ref.at[j, :, pl.ds(h * HALF, HALF)],
                send_sem=send_sems.at[6],
                recv_sem=recv_sems.at[2 * j + h],
                device_id=(src,),
                device_id_type=pl.DeviceIdType.MESH,
            ).wait_recv()
            if state["cp"] is not None:
                state["cp"].wait()
            stage_ref[...] = recv_ref[j, :, pl.ds(h * HALF, HALF)].astype(
                jnp.float32)
            cp = pltpu.make_async_copy(
                stage_ref,
                out_ref.at[pl.ds(src * m_per, m_per), pl.ds(h * HALF, HALF)],
                out_sems.at[2])
            cp.start()
            state["cp"] = cp

        for jj in range(6):
            compute_job(jj)
        compute_job(6)
        drain(1, 0)
        drain(1, 1)
        own_cps[0].wait()
        compute_job(7)
        for j, h in ((0, 0), (0, 1), (2, 0), (2, 1)):
            drain(j, h)

        for jj in (2, 3, 4, 5):
            rdmas[jj].wait_send()
        own_cps[1].wait()
        state["cp"].wait()

    return pl.pallas_call(
        body,
        out_shape=jax.ShapeDtypeStruct((N_DEV * m_per, n_per), jnp.float32),
        in_specs=[
            pl.BlockSpec(memory_space=pltpu.HBM),
            pl.BlockSpec(memory_space=pltpu.HBM),
        ],
        out_specs=pl.BlockSpec(memory_space=pltpu.HBM),
        scratch_shapes=[
            pltpu.VMEM((2, m_per, X_TILE), jnp.float32),
            pltpu.VMEM((m_per, k_dim), jnp.bfloat16),
            pltpu.VMEM((2, K_CHUNK, HALF), jnp.float32),
            pltpu.VMEM((m_per, HALF), jnp.float32),
            pltpu.VMEM((m_per, HALF), jnp.float32),
            pltpu.VMEM((4, m_per, HALF), jnp.bfloat16),
            pltpu.VMEM((3, m_per, n_per), jnp.bfloat16),
            pltpu.SemaphoreType.DMA((2,)),
            pltpu.SemaphoreType.DMA((2,)),
            pltpu.SemaphoreType.DMA((7,)),
            pltpu.SemaphoreType.DMA((6,)),
            pltpu.SemaphoreType.DMA((3,)),
        ],
        compiler_params=pltpu.CompilerParams(
            collective_id=0,
            vmem_limit_bytes=64 * 1024 * 1024,
        ),
    )(x, w_mat)
